# baseline (speedup 1.0000x reference)
import functools

import jax
import jax.numpy as jnp
from jax import lax
from jax.experimental import pallas as pl
from jax.experimental.pallas import tpu as pltpu
from jax.experimental.pallas import tpu_sc as plsc

NUM_EXPERTS = 8
TOP_K = 2
NPAIR = 28
T = 8192
D = 768
DW = D // 2
BB = 128
NSUB = 4
NPAD = T + NPAIR * BB
NB = NPAD // BB
RB = 1024
NRB = T // RB
NEG = -1e30

NW = 32
TPW = T // NW
CH = 64
NCH = TPW // CH


def _round_f16(x):
    r = jax.lax.bitcast_convert_type(x, jnp.int32)
    r = (r + 0x0FFF + ((r >> 13) & 1)) & ~0x1FFF
    return jax.lax.bitcast_convert_type(r, jnp.float32)


def _router_block(flat_ref, gate_ref, idx_ref, gates_ref, load_ref, dst_ref,
                  bmap_ref, flatbf_ref, cnt_sc, base_sc, counts_sc):
    i = pl.program_id(0)

    @pl.when(i < NRB)
    def _phase_a():
        x = flat_ref[...]
        xr = jax.lax.bitcast_convert_type(
            x.astype(jnp.bfloat16).astype(jnp.float32), jnp.int32)
        lo = jax.lax.shift_right_logical(xr[:, :DW], 16)
        hi = xr[:, DW:] & jnp.int32(-65536)
        flatbf_ref[...] = jax.lax.bitcast_convert_type(lo | hi, jnp.float32)
        g = jnp.concatenate(
            [gate_ref[...], jnp.zeros((D, 128 - NUM_EXPERTS), jnp.float32)],
            axis=1)
        logits = jnp.dot(x, g, preferred_element_type=jnp.float32)
        col = jax.lax.broadcasted_iota(jnp.int32, logits.shape, 1)
        logits = jnp.where(col < NUM_EXPERTS, logits, NEG)

        v1 = jnp.max(logits, axis=1, keepdims=True)
        i1 = jnp.min(jnp.where(logits == v1, col, 128), axis=1, keepdims=True)
        l2 = jnp.where(col == i1, NEG, logits)
        v2 = jnp.max(l2, axis=1, keepdims=True)
        i2 = jnp.min(jnp.where(l2 == v2, col, 128), axis=1, keepdims=True)

        e2v = jnp.exp(v2 - v1)
        g1 = _round_f16(1.0 / (1.0 + e2v))
        g2 = _round_f16(e2v / (1.0 + e2v))

        gates_blk = jnp.where(col == i1, g1, 0.0) + jnp.where(col == i2, g2, 0.0)
        gates_ref[...] = gates_blk[:, :NUM_EXPERTS]
        idx_ref[...] = jnp.concatenate([i1, i2], axis=1)

        @pl.when(i == 0)
        def _():
            load_ref[...] = jnp.zeros_like(load_ref)
            counts_sc[...] = jnp.zeros_like(counts_sc)

        load_ref[...] += jnp.sum((gates_blk > 0).astype(jnp.int32), axis=0,
                                 keepdims=True)

        pa = jnp.minimum(i1, i2)
        pb = jnp.maximum(i1, i2)
        pair = (pa * (15 - pa)) // 2 + (pb - pa - 1)
        cnt = (col == pair).astype(jnp.float32)

        r_iota = jax.lax.broadcasted_iota(jnp.int32, (RB, RB), 0)
        c_iota = jax.lax.broadcasted_iota(jnp.int32, (RB, RB), 1)
        tri = (c_iota < r_iota).astype(jnp.bfloat16)
        excl = jnp.dot(tri, cnt.astype(jnp.bfloat16),
                       preferred_element_type=jnp.float32)
        base = excl + counts_sc[...]

        cnt_sc[pl.ds(i * RB, RB), :] = cnt
        base_sc[pl.ds(i * RB, RB), :] = base
        counts_sc[...] += jnp.sum(cnt, axis=0, keepdims=True)

    @pl.when(i == NRB)
    def _phase_b_setup():
        c = counts_sc[...]
        rb = jnp.floor((c + (BB - 1)) / BB)
        k_iota = jax.lax.broadcasted_iota(jnp.int32, (128, 128), 0)
        p_iota = jax.lax.broadcasted_iota(jnp.int32, (128, 128), 1)
        tri = (k_iota < p_iota).astype(jnp.bfloat16)
        excl_off = jnp.dot(rb.astype(jnp.bfloat16), tri,
                           preferred_element_type=jnp.float32) * BB
        r = rb * BB

        incl = excl_off + r
        incl_mat = jnp.broadcast_to(incl, (128, 128))
        bstart = (jax.lax.broadcasted_iota(jnp.int32, (128, 128), 0)
                  .astype(jnp.float32) * BB)
        used = ((incl_mat <= bstart) & (p_iota < NPAIR)).astype(jnp.float32)
        bmap = jnp.sum(used, axis=1, keepdims=True)
        bmap = jnp.minimum(bmap, NPAIR - 1).astype(jnp.int32)
        bmap_ref[...] = jnp.broadcast_to(bmap, (128, 128))

        cnt = cnt_sc[...]
        base = base_sc[...]
        dst = jnp.sum(cnt * (excl_off + base), axis=1)
        dst_ref[...] = jnp.reshape(dst.astype(jnp.int32), (T // 128, 128))


def _router(flat, gate_pad):
    return pl.pallas_call(
        _router_block,
        grid=(NRB + 1,),
        in_specs=[
            pl.BlockSpec((RB, D), lambda i: (jnp.minimum(i, NRB - 1), 0)),
            pl.BlockSpec((D, NUM_EXPERTS), lambda i: (0, 0)),
        ],
        out_specs=[
            pl.BlockSpec((RB, TOP_K), lambda i: (jnp.minimum(i, NRB - 1), 0)),
            pl.BlockSpec((RB, NUM_EXPERTS), lambda i: (jnp.minimum(i, NRB - 1), 0)),
            pl.BlockSpec((1, 128), lambda i: (0, 0)),
            pl.BlockSpec((T // 128, 128), lambda i: (0, 0)),
            pl.BlockSpec((128, 128), lambda i: (0, 0)),
            pl.BlockSpec((RB, DW), lambda i: (jnp.minimum(i, NRB - 1), 0)),
        ],
        out_shape=[
            jax.ShapeDtypeStruct((T, TOP_K), jnp.int32),
            jax.ShapeDtypeStruct((T, NUM_EXPERTS), jnp.float32),
            jax.ShapeDtypeStruct((1, 128), jnp.int32),
            jax.ShapeDtypeStruct((T // 128, 128), jnp.int32),
            jax.ShapeDtypeStruct((128, 128), jnp.int32),
            jax.ShapeDtypeStruct((T, DW), jnp.float32),
        ],
        scratch_shapes=[
            pltpu.VMEM((T, 128), jnp.float32),
            pltpu.VMEM((T, 128), jnp.float32),
            pltpu.VMEM((1, 128), jnp.float32),
        ],
    )(flat, gate_pad)


@functools.cache
def _sc_kernels():
    mesh = plsc.VectorSubcoreMesh(core_axis_name="c", subcore_axis_name="s")

    @functools.partial(
        pl.kernel,
        out_type=jax.ShapeDtypeStruct((NPAD, DW), jnp.float32),
        mesh=mesh,
        scratch_types=[
            pltpu.VMEM((128,), jnp.int32),
            pltpu.VMEM((128,), jnp.int32),
            pltpu.VMEM((128, DW), jnp.float32),
            pltpu.VMEM((128, DW), jnp.float32),
            pltpu.SemaphoreType.DMA,
            pltpu.SemaphoreType.DMA,
            pltpu.SemaphoreType.DMA,
            pltpu.SemaphoreType.DMA,
        ],
    )
    def dispatch_sc(flat, dst_o, xg, idx0, idx1, rows0, rows1, si0, si1,
                    so0, so1):
        wid = lax.axis_index("s") * 2 + lax.axis_index("c")
        base = wid * TPW
        idxs, rows, sis, sos = (idx0, idx1), (rows0, rows1), (si0, si1), (so0, so1)
        pltpu.sync_copy(dst_o.at[2 * wid], idxs[0])
        pltpu.sync_copy(dst_o.at[2 * wid + 1], idxs[1])
        in0 = pltpu.async_copy(flat.at[pl.ds(base, 128)], rows[0], sis[0])
        in1 = pltpu.async_copy(flat.at[pl.ds(base + 128, 128)], rows[1], sis[1])
        in0.wait()
        out0 = pltpu.async_copy(rows[0], xg.at[idxs[0]], sos[0])
        in1.wait()
        out1 = pltpu.async_copy(rows[1], xg.at[idxs[1]], sos[1])
        out0.wait()
        out1.wait()

    @functools.partial(
        pl.kernel,
        out_type=jax.ShapeDtypeStruct((T, D), jnp.float32),
        mesh=mesh,
        scratch_types=[
            pltpu.VMEM((2, 128), jnp.int32),
            pltpu.VMEM((CH, D), jnp.float32),
            pltpu.VMEM((CH, D), jnp.float32),
            pltpu.SemaphoreType.DMA,
            pltpu.SemaphoreType.DMA,
            pltpu.SemaphoreType.DMA,
            pltpu.SemaphoreType.DMA,
        ],
    )
    def combine_sc(y, dst_o, res, idx_v, rows0, rows1, si0, si1, so0, so1):
        wid = lax.axis_index("s") * 2 + lax.axis_index("c")
        base = wid * TPW
        rows, sis, sos = (rows0, rows1), (si0, si1), (so0, so1)
        pltpu.sync_copy(dst_o.at[pl.ds(2 * wid, 2)], idx_v)
        ins = [None, None]
        outs = [None, None]
        ins[0] = pltpu.async_copy(
            y.at[idx_v.at[0, pl.ds(0, CH)]], rows[0], sis[0])
        for c in range(NCH):
            k, nk = c % 2, (c + 1) % 2
            if c + 1 < NCH:
                if outs[nk] is not None:
                    outs[nk].wait()
                c1 = c + 1
                ins[nk] = pltpu.async_copy(
                    y.at[idx_v.at[c1 // 2, pl.ds(CH * (c1 % 2), CH)]],
                    rows[nk], sis[nk])
            ins[k].wait()
            outs[k] = pltpu.async_copy(rows[k], res.at[pl.ds(base + c * CH, CH)],
                                       sos[k])
        outs[0].wait()
        outs[1].wait()

    return dispatch_sc, combine_sc


_PAIRS = [(a, b) for a in range(NUM_EXPERTS) for b in range(a + 1, NUM_EXPERTS)]


def _mm_block(bmap_ref, x_ref, w_ref, y_ref, ws_sc):
    b = pl.program_id(0)

    @pl.when(b == 0)
    def _():
        for p, (ea, eb) in enumerate(_PAIRS):
            ws_sc[p] = w_ref[ea] + w_ref[eb]

    w = jax.lax.bitcast_convert_type(x_ref[...], jnp.int32)
    xa = jax.lax.bitcast_convert_type(jax.lax.shift_left(w, 16), jnp.float32)
    xb = jax.lax.bitcast_convert_type(w & jnp.int32(-65536), jnp.float32)
    x = jnp.concatenate([xa, xb], axis=1).astype(jnp.bfloat16)
    for s in range(NSUB):
        p = bmap_ref[b * NSUB + s]
        y_ref[pl.ds(s * BB, BB), :] = jnp.dot(
            x[s * BB:(s + 1) * BB], ws_sc[pl.ds(p, 1)][0],
            preferred_element_type=jnp.float32)


def _matmul(bmap, xg, w):
    return pl.pallas_call(
        _mm_block,
        grid=(NB // NSUB,),
        in_specs=[
            pl.BlockSpec(memory_space=pltpu.SMEM),
            pl.BlockSpec((NSUB * BB, DW), lambda b: (b, 0)),
            pl.BlockSpec((NUM_EXPERTS, D, D), lambda b: (0, 0, 0)),
        ],
        out_specs=pl.BlockSpec((NSUB * BB, D), lambda b: (b, 0)),
        scratch_shapes=[pltpu.VMEM((NPAIR, D, D), jnp.bfloat16)],
        out_shape=jax.ShapeDtypeStruct((NPAD, D), jnp.float32),
    )(bmap, xg, w)


def kernel(inputs, clean_gate, noise_gate, expert_W, patch_h, patch_w):
    b, s, dim = inputs.shape
    flat = inputs.reshape(-1, dim)

    idx_o, gates_o, load_o, dst_o, bmap_o, flat_bf = _router(flat, clean_gate)

    bmap = bmap_o[:NB, 0]

    dispatch_sc, combine_sc = _sc_kernels()
    xg = dispatch_sc(flat_bf, dst_o)
    y = _matmul(bmap, xg, expert_W.astype(jnp.bfloat16))
    res = combine_sc(y, dst_o)

    return (res.reshape(b, s, D), idx_o, gates_o, load_o[0, :NUM_EXPERTS])


# speedup vs baseline: 1.0491x; 1.0491x over previous
import functools

import jax
import jax.numpy as jnp
from jax import lax
from jax.experimental import pallas as pl
from jax.experimental.pallas import tpu as pltpu
from jax.experimental.pallas import tpu_sc as plsc

NUM_EXPERTS = 8
TOP_K = 2
NPAIR = 28
T = 8192
D = 768
DW = D // 2
BB = 128
NSUB = 8
NPAD = T + 4096
NB = NPAD // BB
RB = 1024
NRB = T // RB
NEG = -1e30

NW = 32
TPW = T // NW
CH = 64
NCH = TPW // CH


def _round_f16(x):
    r = jax.lax.bitcast_convert_type(x, jnp.int32)
    r = (r + 0x0FFF + ((r >> 13) & 1)) & ~0x1FFF
    return jax.lax.bitcast_convert_type(r, jnp.float32)


def _router_block(flat_ref, gate_ref, idx_ref, gates_ref, load_ref, dst_ref,
                  bmap_ref, flatbf_ref, cnt_sc, base_sc, counts_sc):
    i = pl.program_id(0)

    @pl.when(i < NRB)
    def _phase_a():
        x = flat_ref[...]
        xr = jax.lax.bitcast_convert_type(
            x.astype(jnp.bfloat16).astype(jnp.float32), jnp.int32)
        lo = jax.lax.shift_right_logical(xr[:, :DW], 16)
        hi = xr[:, DW:] & jnp.int32(-65536)
        flatbf_ref[...] = jax.lax.bitcast_convert_type(lo | hi, jnp.float32)
        g = jnp.concatenate(
            [gate_ref[...], jnp.zeros((D, 128 - NUM_EXPERTS), jnp.float32)],
            axis=1)
        logits = jnp.dot(x, g, preferred_element_type=jnp.float32)
        col = jax.lax.broadcasted_iota(jnp.int32, logits.shape, 1)
        logits = jnp.where(col < NUM_EXPERTS, logits, NEG)

        v1 = jnp.max(logits, axis=1, keepdims=True)
        i1 = jnp.min(jnp.where(logits == v1, col, 128), axis=1, keepdims=True)
        l2 = jnp.where(col == i1, NEG, logits)
        v2 = jnp.max(l2, axis=1, keepdims=True)
        i2 = jnp.min(jnp.where(l2 == v2, col, 128), axis=1, keepdims=True)

        e2v = jnp.exp(v2 - v1)
        g1 = _round_f16(1.0 / (1.0 + e2v))
        g2 = _round_f16(e2v / (1.0 + e2v))

        gates_blk = jnp.where(col == i1, g1, 0.0) + jnp.where(col == i2, g2, 0.0)
        gates_ref[...] = gates_blk[:, :NUM_EXPERTS]
        idx_ref[...] = jnp.concatenate([i1, i2], axis=1)

        @pl.when(i == 0)
        def _():
            load_ref[...] = jnp.zeros_like(load_ref)
            counts_sc[...] = jnp.zeros_like(counts_sc)

        load_ref[...] += jnp.sum((gates_blk > 0).astype(jnp.int32), axis=0,
                                 keepdims=True)

        pa = jnp.minimum(i1, i2)
        pb = jnp.maximum(i1, i2)
        pair = (pa * (15 - pa)) // 2 + (pb - pa - 1)
        cnt = (col == pair).astype(jnp.float32)

        r_iota = jax.lax.broadcasted_iota(jnp.int32, (RB, RB), 0)
        c_iota = jax.lax.broadcasted_iota(jnp.int32, (RB, RB), 1)
        tri = (c_iota < r_iota).astype(jnp.bfloat16)
        excl = jnp.dot(tri, cnt.astype(jnp.bfloat16),
                       preferred_element_type=jnp.float32)
        base = excl + counts_sc[...]

        cnt_sc[pl.ds(i * RB, RB), :] = cnt
        base_sc[pl.ds(i * RB, RB), :] = base
        counts_sc[...] += jnp.sum(cnt, axis=0, keepdims=True)

    @pl.when(i == NRB)
    def _phase_b_setup():
        c = counts_sc[...]
        rb = jnp.floor((c + (BB - 1)) / BB)
        k_iota = jax.lax.broadcasted_iota(jnp.int32, (128, 128), 0)
        p_iota = jax.lax.broadcasted_iota(jnp.int32, (128, 128), 1)
        tri = (k_iota < p_iota).astype(jnp.bfloat16)
        excl_off = jnp.dot(rb.astype(jnp.bfloat16), tri,
                           preferred_element_type=jnp.float32) * BB
        r = rb * BB

        incl = excl_off + r
        incl_mat = jnp.broadcast_to(incl, (128, 128))
        bstart = (jax.lax.broadcasted_iota(jnp.int32, (128, 128), 0)
                  .astype(jnp.float32) * BB)
        used = ((incl_mat <= bstart) & (p_iota < NPAIR)).astype(jnp.float32)
        bmap = jnp.sum(used, axis=1, keepdims=True)
        bmap = jnp.minimum(bmap, NPAIR - 1).astype(jnp.int32)
        bmap_ref[...] = jnp.broadcast_to(bmap, (128, 128))

        cnt = cnt_sc[...]
        base = base_sc[...]
        dst = jnp.sum(cnt * (excl_off + base), axis=1)
        dst_ref[...] = jnp.reshape(dst.astype(jnp.int32), (T // 128, 128))


def _router(flat, gate_pad):
    return pl.pallas_call(
        _router_block,
        grid=(NRB + 1,),
        in_specs=[
            pl.BlockSpec((RB, D), lambda i: (jnp.minimum(i, NRB - 1), 0)),
            pl.BlockSpec((D, NUM_EXPERTS), lambda i: (0, 0)),
        ],
        out_specs=[
            pl.BlockSpec((RB, TOP_K), lambda i: (jnp.minimum(i, NRB - 1), 0)),
            pl.BlockSpec((RB, NUM_EXPERTS), lambda i: (jnp.minimum(i, NRB - 1), 0)),
            pl.BlockSpec((1, 128), lambda i: (0, 0)),
            pl.BlockSpec((T // 128, 128), lambda i: (0, 0)),
            pl.BlockSpec((128, 128), lambda i: (0, 0)),
            pl.BlockSpec((RB, DW), lambda i: (jnp.minimum(i, NRB - 1), 0)),
        ],
        out_shape=[
            jax.ShapeDtypeStruct((T, TOP_K), jnp.int32),
            jax.ShapeDtypeStruct((T, NUM_EXPERTS), jnp.float32),
            jax.ShapeDtypeStruct((1, 128), jnp.int32),
            jax.ShapeDtypeStruct((T // 128, 128), jnp.int32),
            jax.ShapeDtypeStruct((128, 128), jnp.int32),
            jax.ShapeDtypeStruct((T, DW), jnp.float32),
        ],
        scratch_shapes=[
            pltpu.VMEM((T, 128), jnp.float32),
            pltpu.VMEM((T, 128), jnp.float32),
            pltpu.VMEM((1, 128), jnp.float32),
        ],
    )(flat, gate_pad)


@functools.cache
def _sc_kernels():
    mesh = plsc.VectorSubcoreMesh(core_axis_name="c", subcore_axis_name="s")

    @functools.partial(
        pl.kernel,
        out_type=jax.ShapeDtypeStruct((NPAD, DW), jnp.float32),
        mesh=mesh,
        scratch_types=[
            pltpu.VMEM((128,), jnp.int32),
            pltpu.VMEM((128,), jnp.int32),
            pltpu.VMEM((128, DW), jnp.float32),
            pltpu.VMEM((128, DW), jnp.float32),
            pltpu.SemaphoreType.DMA,
            pltpu.SemaphoreType.DMA,
            pltpu.SemaphoreType.DMA,
            pltpu.SemaphoreType.DMA,
        ],
    )
    def dispatch_sc(flat, dst_o, xg, idx0, idx1, rows0, rows1, si0, si1,
                    so0, so1):
        wid = lax.axis_index("s") * 2 + lax.axis_index("c")
        base = wid * TPW
        idxs, rows, sis, sos = (idx0, idx1), (rows0, rows1), (si0, si1), (so0, so1)
        pltpu.sync_copy(dst_o.at[2 * wid], idxs[0])
        pltpu.sync_copy(dst_o.at[2 * wid + 1], idxs[1])
        in0 = pltpu.async_copy(flat.at[pl.ds(base, 128)], rows[0], sis[0])
        in1 = pltpu.async_copy(flat.at[pl.ds(base + 128, 128)], rows[1], sis[1])
        in0.wait()
        out0 = pltpu.async_copy(rows[0], xg.at[idxs[0]], sos[0])
        in1.wait()
        out1 = pltpu.async_copy(rows[1], xg.at[idxs[1]], sos[1])
        out0.wait()
        out1.wait()

    @functools.partial(
        pl.kernel,
        out_type=jax.ShapeDtypeStruct((T, D), jnp.float32),
        mesh=mesh,
        scratch_types=[
            pltpu.VMEM((2, 128), jnp.int32),
            pltpu.VMEM((CH, D), jnp.float32),
            pltpu.VMEM((CH, D), jnp.float32),
            pltpu.SemaphoreType.DMA,
            pltpu.SemaphoreType.DMA,
            pltpu.SemaphoreType.DMA,
            pltpu.SemaphoreType.DMA,
        ],
    )
    def combine_sc(y, dst_o, res, idx_v, rows0, rows1, si0, si1, so0, so1):
        wid = lax.axis_index("s") * 2 + lax.axis_index("c")
        base = wid * TPW
        rows, sis, sos = (rows0, rows1), (si0, si1), (so0, so1)
        pltpu.sync_copy(dst_o.at[pl.ds(2 * wid, 2)], idx_v)
        ins = [None, None]
        outs = [None, None]
        ins[0] = pltpu.async_copy(
            y.at[idx_v.at[0, pl.ds(0, CH)]], rows[0], sis[0])
        for c in range(NCH):
            k, nk = c % 2, (c + 1) % 2
            if c + 1 < NCH:
                if outs[nk] is not None:
                    outs[nk].wait()
                c1 = c + 1
                ins[nk] = pltpu.async_copy(
                    y.at[idx_v.at[c1 // 2, pl.ds(CH * (c1 % 2), CH)]],
                    rows[nk], sis[nk])
            ins[k].wait()
            outs[k] = pltpu.async_copy(rows[k], res.at[pl.ds(base + c * CH, CH)],
                                       sos[k])
        outs[0].wait()
        outs[1].wait()

    return dispatch_sc, combine_sc


_PAIRS = [(a, b) for a in range(NUM_EXPERTS) for b in range(a + 1, NUM_EXPERTS)]


def _mm_block(bmap_ref, x_ref, w_ref, y_ref, ws_sc):
    b = pl.program_id(0)

    @pl.when(b == 0)
    def _():
        for p, (ea, eb) in enumerate(_PAIRS):
            ws_sc[p] = w_ref[ea] + w_ref[eb]

    w = jax.lax.bitcast_convert_type(x_ref[...], jnp.int32)
    xa = jax.lax.bitcast_convert_type(jax.lax.shift_left(w, 16), jnp.float32)
    xb = jax.lax.bitcast_convert_type(w & jnp.int32(-65536), jnp.float32)
    x = jnp.concatenate([xa, xb], axis=1).astype(jnp.bfloat16)
    for s in range(NSUB):
        p = bmap_ref[b * NSUB + s]
        y_ref[pl.ds(s * BB, BB), :] = jnp.dot(
            x[s * BB:(s + 1) * BB], ws_sc[pl.ds(p, 1)][0],
            preferred_element_type=jnp.float32)


def _matmul(bmap, xg, w):
    return pl.pallas_call(
        _mm_block,
        grid=(NB // NSUB,),
        in_specs=[
            pl.BlockSpec(memory_space=pltpu.SMEM),
            pl.BlockSpec((NSUB * BB, DW), lambda b: (b, 0)),
            pl.BlockSpec((NUM_EXPERTS, D, D), lambda b: (0, 0, 0)),
        ],
        out_specs=pl.BlockSpec((NSUB * BB, D), lambda b: (b, 0)),
        scratch_shapes=[pltpu.VMEM((NPAIR, D, D), jnp.bfloat16)],
        out_shape=jax.ShapeDtypeStruct((NPAD, D), jnp.float32),
    )(bmap, xg, w)


def kernel(inputs, clean_gate, noise_gate, expert_W, patch_h, patch_w):
    b, s, dim = inputs.shape
    flat = inputs.reshape(-1, dim)

    idx_o, gates_o, load_o, dst_o, bmap_o, flat_bf = _router(flat, clean_gate)

    bmap = bmap_o[:NB, 0]

    dispatch_sc, combine_sc = _sc_kernels()
    xg = dispatch_sc(flat_bf, dst_o)
    y = _matmul(bmap, xg, expert_W.astype(jnp.bfloat16))
    res = combine_sc(y, dst_o)

    return (res.reshape(b, s, D), idx_o, gates_o, load_o[0, :NUM_EXPERTS])


# speedup vs baseline: 1.0673x; 1.0173x over previous
import functools

import jax
import jax.numpy as jnp
from jax import lax
from jax.experimental import pallas as pl
from jax.experimental.pallas import tpu as pltpu
from jax.experimental.pallas import tpu_sc as plsc

NUM_EXPERTS = 8
TOP_K = 2
NPAIR = 28
T = 8192
D = 768
DW = D // 2
BB = 128
NSUB = 12
NPAD = T + 4096
NB = NPAD // BB
RB = 1024
NRB = T // RB
NEG = -1e30

NW = 32
TPW = T // NW
CH = 64
NCH = TPW // CH


def _round_f16(x):
    r = jax.lax.bitcast_convert_type(x, jnp.int32)
    r = (r + 0x0FFF + ((r >> 13) & 1)) & ~0x1FFF
    return jax.lax.bitcast_convert_type(r, jnp.float32)


def _router_block(flat_ref, gate_ref, idx_ref, gates_ref, load_ref, dst_ref,
                  bmap_ref, flatbf_ref, cnt_sc, base_sc, counts_sc):
    i = pl.program_id(0)

    @pl.when(i < NRB)
    def _phase_a():
        x = flat_ref[...]
        xr = jax.lax.bitcast_convert_type(
            x.astype(jnp.bfloat16).astype(jnp.float32), jnp.int32)
        lo = jax.lax.shift_right_logical(xr[:, :DW], 16)
        hi = xr[:, DW:] & jnp.int32(-65536)
        flatbf_ref[...] = jax.lax.bitcast_convert_type(lo | hi, jnp.float32)
        g = jnp.concatenate(
            [gate_ref[...], jnp.zeros((D, 128 - NUM_EXPERTS), jnp.float32)],
            axis=1)
        logits = jnp.dot(x, g, preferred_element_type=jnp.float32)
        col = jax.lax.broadcasted_iota(jnp.int32, logits.shape, 1)
        logits = jnp.where(col < NUM_EXPERTS, logits, NEG)

        v1 = jnp.max(logits, axis=1, keepdims=True)
        i1 = jnp.min(jnp.where(logits == v1, col, 128), axis=1, keepdims=True)
        l2 = jnp.where(col == i1, NEG, logits)
        v2 = jnp.max(l2, axis=1, keepdims=True)
        i2 = jnp.min(jnp.where(l2 == v2, col, 128), axis=1, keepdims=True)

        e2v = jnp.exp(v2 - v1)
        g1 = _round_f16(1.0 / (1.0 + e2v))
        g2 = _round_f16(e2v / (1.0 + e2v))

        gates_blk = jnp.where(col == i1, g1, 0.0) + jnp.where(col == i2, g2, 0.0)
        gates_ref[...] = gates_blk[:, :NUM_EXPERTS]
        idx_ref[...] = jnp.concatenate([i1, i2], axis=1)

        @pl.when(i == 0)
        def _():
            load_ref[...] = jnp.zeros_like(load_ref)
            counts_sc[...] = jnp.zeros_like(counts_sc)

        load_ref[...] += jnp.sum((gates_blk > 0).astype(jnp.int32), axis=0,
                                 keepdims=True)

        pa = jnp.minimum(i1, i2)
        pb = jnp.maximum(i1, i2)
        pair = (pa * (15 - pa)) // 2 + (pb - pa - 1)
        cnt = (col == pair).astype(jnp.float32)

        r_iota = jax.lax.broadcasted_iota(jnp.int32, (RB, RB), 0)
        c_iota = jax.lax.broadcasted_iota(jnp.int32, (RB, RB), 1)
        tri = (c_iota < r_iota).astype(jnp.bfloat16)
        excl = jnp.dot(tri, cnt.astype(jnp.bfloat16),
                       preferred_element_type=jnp.float32)
        base = excl + counts_sc[...]

        cnt_sc[pl.ds(i * RB, RB), :] = cnt
        base_sc[pl.ds(i * RB, RB), :] = base
        counts_sc[...] += jnp.sum(cnt, axis=0, keepdims=True)

    @pl.when(i == NRB)
    def _phase_b_setup():
        c = counts_sc[...]
        rb = jnp.floor((c + (BB - 1)) / BB)
        k_iota = jax.lax.broadcasted_iota(jnp.int32, (128, 128), 0)
        p_iota = jax.lax.broadcasted_iota(jnp.int32, (128, 128), 1)
        tri = (k_iota < p_iota).astype(jnp.bfloat16)
        excl_off = jnp.dot(rb.astype(jnp.bfloat16), tri,
                           preferred_element_type=jnp.float32) * BB
        r = rb * BB

        incl = excl_off + r
        incl_mat = jnp.broadcast_to(incl, (128, 128))
        bstart = (jax.lax.broadcasted_iota(jnp.int32, (128, 128), 0)
                  .astype(jnp.float32) * BB)
        used = ((incl_mat <= bstart) & (p_iota < NPAIR)).astype(jnp.float32)
        bmap = jnp.sum(used, axis=1, keepdims=True)
        bmap = jnp.minimum(bmap, NPAIR - 1).astype(jnp.int32)
        bmap_ref[...] = jnp.broadcast_to(bmap, (128, 128))

        cnt = cnt_sc[...]
        base = base_sc[...]
        dst = jnp.sum(cnt * (excl_off + base), axis=1)
        dst_ref[...] = jnp.reshape(dst.astype(jnp.int32), (T // 128, 128))


def _router(flat, gate_pad):
    return pl.pallas_call(
        _router_block,
        grid=(NRB + 1,),
        in_specs=[
            pl.BlockSpec((RB, D), lambda i: (jnp.minimum(i, NRB - 1), 0)),
            pl.BlockSpec((D, NUM_EXPERTS), lambda i: (0, 0)),
        ],
        out_specs=[
            pl.BlockSpec((RB, TOP_K), lambda i: (jnp.minimum(i, NRB - 1), 0)),
            pl.BlockSpec((RB, NUM_EXPERTS), lambda i: (jnp.minimum(i, NRB - 1), 0)),
            pl.BlockSpec((1, 128), lambda i: (0, 0)),
            pl.BlockSpec((T // 128, 128), lambda i: (0, 0)),
            pl.BlockSpec((128, 128), lambda i: (0, 0)),
            pl.BlockSpec((RB, DW), lambda i: (jnp.minimum(i, NRB - 1), 0)),
        ],
        out_shape=[
            jax.ShapeDtypeStruct((T, TOP_K), jnp.int32),
            jax.ShapeDtypeStruct((T, NUM_EXPERTS), jnp.float32),
            jax.ShapeDtypeStruct((1, 128), jnp.int32),
            jax.ShapeDtypeStruct((T // 128, 128), jnp.int32),
            jax.ShapeDtypeStruct((128, 128), jnp.int32),
            jax.ShapeDtypeStruct((T, DW), jnp.float32),
        ],
        scratch_shapes=[
            pltpu.VMEM((T, 128), jnp.float32),
            pltpu.VMEM((T, 128), jnp.float32),
            pltpu.VMEM((1, 128), jnp.float32),
        ],
    )(flat, gate_pad)


@functools.cache
def _sc_kernels():
    mesh = plsc.VectorSubcoreMesh(core_axis_name="c", subcore_axis_name="s")

    @functools.partial(
        pl.kernel,
        out_type=jax.ShapeDtypeStruct((NPAD, DW), jnp.float32),
        mesh=mesh,
        scratch_types=[
            pltpu.VMEM((128,), jnp.int32),
            pltpu.VMEM((128,), jnp.int32),
            pltpu.VMEM((128, DW), jnp.float32),
            pltpu.VMEM((128, DW), jnp.float32),
            pltpu.SemaphoreType.DMA,
            pltpu.SemaphoreType.DMA,
            pltpu.SemaphoreType.DMA,
            pltpu.SemaphoreType.DMA,
        ],
    )
    def dispatch_sc(flat, dst_o, xg, idx0, idx1, rows0, rows1, si0, si1,
                    so0, so1):
        wid = lax.axis_index("s") * 2 + lax.axis_index("c")
        base = wid * TPW
        idxs, rows, sis, sos = (idx0, idx1), (rows0, rows1), (si0, si1), (so0, so1)
        pltpu.sync_copy(dst_o.at[2 * wid], idxs[0])
        pltpu.sync_copy(dst_o.at[2 * wid + 1], idxs[1])
        in0 = pltpu.async_copy(flat.at[pl.ds(base, 128)], rows[0], sis[0])
        in1 = pltpu.async_copy(flat.at[pl.ds(base + 128, 128)], rows[1], sis[1])
        in0.wait()
        out0 = pltpu.async_copy(rows[0], xg.at[idxs[0]], sos[0])
        in1.wait()
        out1 = pltpu.async_copy(rows[1], xg.at[idxs[1]], sos[1])
        out0.wait()
        out1.wait()

    @functools.partial(
        pl.kernel,
        out_type=jax.ShapeDtypeStruct((T, D), jnp.float32),
        mesh=mesh,
        scratch_types=[
            pltpu.VMEM((2, 128), jnp.int32),
            pltpu.VMEM((CH, D), jnp.float32),
            pltpu.VMEM((CH, D), jnp.float32),
            pltpu.SemaphoreType.DMA,
            pltpu.SemaphoreType.DMA,
            pltpu.SemaphoreType.DMA,
            pltpu.SemaphoreType.DMA,
        ],
    )
    def combine_sc(y, dst_o, res, idx_v, rows0, rows1, si0, si1, so0, so1):
        wid = lax.axis_index("s") * 2 + lax.axis_index("c")
        base = wid * TPW
        rows, sis, sos = (rows0, rows1), (si0, si1), (so0, so1)
        pltpu.sync_copy(dst_o.at[pl.ds(2 * wid, 2)], idx_v)
        ins = [None, None]
        outs = [None, None]
        ins[0] = pltpu.async_copy(
            y.at[idx_v.at[0, pl.ds(0, CH)]], rows[0], sis[0])
        for c in range(NCH):
            k, nk = c % 2, (c + 1) % 2
            if c + 1 < NCH:
                if outs[nk] is not None:
                    outs[nk].wait()
                c1 = c + 1
                ins[nk] = pltpu.async_copy(
                    y.at[idx_v.at[c1 // 2, pl.ds(CH * (c1 % 2), CH)]],
                    rows[nk], sis[nk])
            ins[k].wait()
            outs[k] = pltpu.async_copy(rows[k], res.at[pl.ds(base + c * CH, CH)],
                                       sos[k])
        outs[0].wait()
        outs[1].wait()

    return dispatch_sc, combine_sc


_PAIRS = [(a, b) for a in range(NUM_EXPERTS) for b in range(a + 1, NUM_EXPERTS)]


def _mm_block(bmap_ref, x_ref, w_ref, y_ref, ws_sc):
    b = pl.program_id(0)

    @pl.when(b == 0)
    def _():
        for p, (ea, eb) in enumerate(_PAIRS):
            ws_sc[p] = w_ref[ea] + w_ref[eb]

    w = jax.lax.bitcast_convert_type(x_ref[...], jnp.int32)
    xa = jax.lax.bitcast_convert_type(jax.lax.shift_left(w, 16), jnp.float32)
    xb = jax.lax.bitcast_convert_type(w & jnp.int32(-65536), jnp.float32)
    x = jnp.concatenate([xa, xb], axis=1).astype(jnp.bfloat16)
    for s in range(NSUB):
        p = bmap_ref[b * NSUB + s]
        y_ref[pl.ds(s * BB, BB), :] = jnp.dot(
            x[s * BB:(s + 1) * BB], ws_sc[pl.ds(p, 1)][0],
            preferred_element_type=jnp.float32)


def _matmul(bmap, xg, w):
    return pl.pallas_call(
        _mm_block,
        grid=(NB // NSUB,),
        in_specs=[
            pl.BlockSpec(memory_space=pltpu.SMEM),
            pl.BlockSpec((NSUB * BB, DW), lambda b: (b, 0)),
            pl.BlockSpec((NUM_EXPERTS, D, D), lambda b: (0, 0, 0)),
        ],
        out_specs=pl.BlockSpec((NSUB * BB, D), lambda b: (b, 0)),
        scratch_shapes=[pltpu.VMEM((NPAIR, D, D), jnp.bfloat16)],
        out_shape=jax.ShapeDtypeStruct((NPAD, D), jnp.float32),
    )(bmap, xg, w)


def kernel(inputs, clean_gate, noise_gate, expert_W, patch_h, patch_w):
    b, s, dim = inputs.shape
    flat = inputs.reshape(-1, dim)

    idx_o, gates_o, load_o, dst_o, bmap_o, flat_bf = _router(flat, clean_gate)

    bmap = bmap_o[:NB, 0]

    dispatch_sc, combine_sc = _sc_kernels()
    xg = dispatch_sc(flat_bf, dst_o)
    y = _matmul(bmap, xg, expert_W.astype(jnp.bfloat16))
    res = combine_sc(y, dst_o)

    return (res.reshape(b, s, D), idx_o, gates_o, load_o[0, :NUM_EXPERTS])
